# baseline (speedup 1.0000x reference)
import jax
import jax.numpy as jnp
import numpy as np
from jax import lax
from jax.experimental import pallas as pl
from jax.experimental.pallas import tpu as pltpu
from jax.experimental.pallas import tpu_sc as plsc

VOCAB = 1000000
D = 64
SEQ = 200
BATCH = 1024
NTOK = BATCH * SEQ
NC, NS, LANES = 2, 16, 16
NW = NC * NS
PER_W = NTOK // NW
CH = 128
NCHUNK = PER_W // CH
SCALE = float(D) ** 0.5


def _pos_encoding_ext() -> np.ndarray:
    positions = np.arange(SEQ).reshape(-1, 1)
    dims = np.arange(D // 2).reshape(1, -1)
    angles = positions / np.power(10000, 2 * dims / D)
    pe = np.zeros((SEQ, D))
    pe[:, 0::2] = np.sin(angles)
    pe[:, 1::2] = np.cos(angles)
    ext = np.concatenate([pe, pe[:CH]], axis=0)
    return ext.astype(np.float32)


_PE_EXT = _pos_encoding_ext()


def _sc_body(x_hbm, table_hbm, pe_hbm, out_hbm, idx_v, buf_v, pe_v, sem):
    wid = lax.axis_index("s") * NC + lax.axis_index("c")
    base = wid * PER_W

    pltpu.sync_copy(pe_hbm, pe_v)

    def chunk_body(g, carry):
        start = base + g * CH
        pltpu.sync_copy(x_hbm.at[pl.ds(start, CH)], idx_v)
        pltpu.async_copy(table_hbm.at[idx_v], buf_v, sem).wait()
        off = lax.rem(g * CH, SEQ)

        def row_body(r, c2):
            for c in range(D // LANES):
                v = buf_v[r, pl.ds(c * LANES, LANES)]
                p = pe_v[off + r, pl.ds(c * LANES, LANES)]
                buf_v[r, pl.ds(c * LANES, LANES)] = v * SCALE + p
            return c2

        lax.fori_loop(0, CH, row_body, 0)
        pltpu.sync_copy(buf_v, out_hbm.at[pl.ds(start, CH)])
        return carry

    lax.fori_loop(0, NCHUNK, chunk_body, 0)


@jax.jit
def _run(x_flat, table, pe_ext):
    mesh = plsc.VectorSubcoreMesh(core_axis_name="c", subcore_axis_name="s")
    f = pl.kernel(
        _sc_body,
        out_type=jax.ShapeDtypeStruct((NTOK, D), jnp.float32),
        mesh=mesh,
        scratch_types=[
            pltpu.VMEM((CH,), jnp.int32),
            pltpu.VMEM((CH, D), jnp.float32),
            pltpu.VMEM((SEQ + CH, D), jnp.float32),
            pltpu.SemaphoreType.DMA,
        ],
        compiler_params=pltpu.CompilerParams(use_tc_tiling_on_sc=False),
    )
    return f(x_flat, table, pe_ext)


def kernel(x, table):
    x_flat = jnp.reshape(x.astype(jnp.int32), (NTOK,))
    pe_ext = jnp.asarray(_PE_EXT)
    out = _run(x_flat, table, pe_ext)
    return jnp.reshape(out, (BATCH, SEQ, D))


# speedup vs baseline: 1.1189x; 1.1189x over previous
import jax
import jax.numpy as jnp
import numpy as np
from jax import lax
from jax.experimental import pallas as pl
from jax.experimental.pallas import tpu as pltpu
from jax.experimental.pallas import tpu_sc as plsc

VOCAB = 1000000
D = 64
SEQ = 200
BATCH = 1024
NTOK = BATCH * SEQ
NC, NS, LANES = 2, 16, 16
NW = NC * NS
PER_W = NTOK // NW
CH = 128
NCHUNK = PER_W // CH
NBUF = 5
NOUT = NCHUNK // NBUF
SCALE = float(D) ** 0.5


def _pos_encoding_ext() -> np.ndarray:
    positions = np.arange(SEQ).reshape(-1, 1)
    dims = np.arange(D // 2).reshape(1, -1)
    angles = positions / np.power(10000, 2 * dims / D)
    pe = np.zeros((SEQ, D))
    pe[:, 0::2] = np.sin(angles)
    pe[:, 1::2] = np.cos(angles)
    ext = np.concatenate([pe, pe[:CH]], axis=0)
    return ext.astype(np.float32)


_PE_EXT = _pos_encoding_ext()


def _sc_body(x_hbm, table_hbm, pe_hbm, out_hbm, idx_v, pe_v, bufs, gsems, ssems):
    wid = lax.axis_index("s") * NC + lax.axis_index("c")
    base = wid * PER_W

    pltpu.sync_copy(pe_hbm, pe_v)
    pltpu.sync_copy(x_hbm.at[wid], idx_v)

    def start_gather(f, slot):
        pltpu.async_copy(table_hbm.at[idx_v.at[f]], bufs[slot], gsems[slot])

    def wait_gather(slot):
        pltpu.make_async_copy(
            table_hbm.at[idx_v.at[0]], bufs[slot], gsems[slot]).wait()

    def start_scatter(g, slot):
        pltpu.async_copy(
            bufs[slot], out_hbm.at[pl.ds(base + g * CH, CH)], ssems[slot])

    def wait_scatter(slot):
        pltpu.make_async_copy(
            bufs[slot], out_hbm.at[pl.ds(base, CH)], ssems[slot]).wait()

    def compute(g, slot):
        off = lax.rem(g * CH, SEQ)
        buf = bufs[slot]

        def row_body(r, carry):
            for c in range(D // LANES):
                v = buf[r, pl.ds(c * LANES, LANES)]
                p = pe_v[off + r, pl.ds(c * LANES, LANES)]
                buf[r, pl.ds(c * LANES, LANES)] = v * SCALE + p
            return carry

        lax.fori_loop(0, CH, row_body, 0, unroll=2)

    for b in range(NBUF - 1):
        start_gather(b, b)

    @pl.loop(0, NOUT)
    def _outer(go):
        for b in range(NBUF):
            g = go * NBUF + b
            f = g + NBUF - 1
            fslot = (b + NBUF - 1) % NBUF

            wait_gather(b)
            compute(g, b)
            start_scatter(g, b)

            @pl.when(f < NCHUNK)
            def _():
                @pl.when(f >= NBUF)
                def _():
                    wait_scatter(fslot)
                start_gather(f, fslot)

    for b in range(NBUF):
        wait_scatter(b)


@jax.jit
def _run(x3, table, pe_ext):
    mesh = plsc.VectorSubcoreMesh(core_axis_name="c", subcore_axis_name="s")
    f = pl.kernel(
        _sc_body,
        out_type=jax.ShapeDtypeStruct((NTOK, D), jnp.float32),
        mesh=mesh,
        scratch_types=[
            pltpu.VMEM((NCHUNK, CH), jnp.int32),
            pltpu.VMEM((SEQ + CH, D), jnp.float32),
            [pltpu.VMEM((CH, D), jnp.float32) for _ in range(NBUF)],
            [pltpu.SemaphoreType.DMA for _ in range(NBUF)],
            [pltpu.SemaphoreType.DMA for _ in range(NBUF)],
        ],
        compiler_params=pltpu.CompilerParams(use_tc_tiling_on_sc=False),
    )
    return f(x3, table, pe_ext)


def kernel(x, table):
    x3 = jnp.reshape(x.astype(jnp.int32), (NW, NCHUNK, CH))
    pe_ext = jnp.asarray(_PE_EXT)
    out = _run(x3, table, pe_ext)
    return jnp.reshape(out, (BATCH, SEQ, D))
